# Initial kernel scaffold; baseline (speedup 1.0000x reference)
import jax
import jax.numpy as jnp
from jax.experimental import pallas as pl


def kernel(x, edge_index, W, b):
    raise NotImplementedError("write your pallas kernel here")


# speedup vs baseline: 8.3603x; 8.3603x over previous
import functools

import jax
import jax.numpy as jnp
from jax import lax
from jax.experimental import pallas as pl
from jax.experimental.pallas import tpu as pltpu
from jax.experimental.pallas import tpu_sc as plsc

N_NODES = 10000
N_EDGES = 320000
D = 128

NC = 2
NS = 16
NW = NC * NS
E_PER_W = N_EDGES // NW
K = 80
CHUNKS = E_PER_W // K
NP = 10240
ROWS_PER_TILE = NP // NS


def _linear(x, Wt, b2):
    def body(x_ref, w_ref, b_ref, o_ref):
        o_ref[...] = (
            jnp.dot(x_ref[...], w_ref[...], preferred_element_type=jnp.float32)
            + b_ref[...]
        )

    return pl.pallas_call(
        body,
        out_shape=jax.ShapeDtypeStruct((N_NODES, D), jnp.float32),
    )(x, Wt, b2)


def _finalize(partials, pdeg):
    def body(p_ref, d_ref, o_ref):
        s = p_ref[0] + p_ref[1]
        deg = jnp.maximum(d_ref[0] + d_ref[1], 1.0)
        o_ref[...] = s / deg[:, None]

    return pl.pallas_call(
        body,
        out_shape=jax.ShapeDtypeStruct((NP, D), jnp.float32),
    )(partials, pdeg)


def _sc_aggregate(h, src_r, dst_r, zrows, zdeg):
    mesh = plsc.VectorSubcoreMesh(core_axis_name="c", subcore_axis_name="s")

    @functools.partial(
        pl.kernel,
        mesh=mesh,
        out_type=[
            jax.ShapeDtypeStruct((NC, NP, D), jnp.float32),
            jax.ShapeDtypeStruct((NC, NP), jnp.float32),
        ],
        scratch_types=[
            pltpu.VMEM((CHUNKS, K), jnp.int32),
            pltpu.VMEM((CHUNKS, K), jnp.int32),
            pltpu.VMEM((K, D), jnp.float32),
            pltpu.VMEM((K,), jnp.float32),
            pltpu.VMEM_SHARED((NP, D), jnp.float32),
            pltpu.VMEM_SHARED((NP,), jnp.float32),
            pltpu.SemaphoreType.DMA,
        ],
    )
    def k(h_hbm, src_hbm, dst_hbm, zrows_hbm, zdeg_hbm, part_hbm, pdeg_hbm,
          sidx_v, didx_v, rows_v, ones_v, acc_sh, deg_sh, sem):
        cid = lax.axis_index("c")
        tid = lax.axis_index("s")
        wid = cid * NS + tid

        r0 = tid * ROWS_PER_TILE
        pltpu.sync_copy(
            zrows_hbm.at[pl.ds(r0, ROWS_PER_TILE)],
            acc_sh.at[pl.ds(r0, ROWS_PER_TILE)],
        )

        @pl.when(tid == 0)
        def _():
            pltpu.sync_copy(zdeg_hbm, deg_sh)

        pltpu.sync_copy(src_hbm.at[wid], sidx_v)
        pltpu.sync_copy(dst_hbm.at[wid], didx_v)

        for i in range(K // 16):
            ones_v[pl.ds(i * 16, 16)] = jnp.full((16,), 1.0, jnp.float32)

        plsc.subcore_barrier()

        def chunk(t, carry):
            pltpu.async_copy(h_hbm.at[sidx_v.at[t]], rows_v, sem).wait()
            pltpu.sync_copy(rows_v, acc_sh.at[didx_v.at[t]], add=True)
            pltpu.sync_copy(ones_v, deg_sh.at[didx_v.at[t]], add=True)
            return carry

        lax.fori_loop(0, CHUNKS, chunk, 0)

        plsc.subcore_barrier()

        pltpu.sync_copy(
            acc_sh.at[pl.ds(r0, ROWS_PER_TILE)],
            part_hbm.at[cid, pl.ds(r0, ROWS_PER_TILE)],
        )

        @pl.when(tid == 0)
        def _():
            pltpu.sync_copy(deg_sh, pdeg_hbm.at[cid])

    return k(h, src_r, dst_r, zrows, zdeg)


def kernel(x, edge_index, W, b):
    ei = edge_index.astype(jnp.int32)
    src_r = ei[0].reshape(NW, CHUNKS, K)
    dst_r = ei[1].reshape(NW, CHUNKS, K)

    h = _linear(x, W.T, b.reshape(1, D))

    zrows = jnp.zeros((NP, D), jnp.float32)
    zdeg = jnp.zeros((NP,), jnp.float32)
    partials, pdeg = _sc_aggregate(h, src_r, dst_r, zrows, zdeg)
    return _finalize(partials, pdeg)[:N_NODES]
